# baseline (speedup 1.0000x reference)
import jax
import jax.numpy as jnp
from jax.experimental import pallas as pl
from jax.experimental.pallas import tpu as pltpu

_F = 26
_D = 100
_DP = 128
_K = 16
_GK = _F * _K


def _ffm_block(x_ref, vr_ref, wp_ref, o_ref, e_ref):
    x = x_ref[...]
    lin = jnp.sum(x * wp_ref[...], axis=(0, 2))

    diag = None
    for f in range(_F):
        ef = jnp.dot(x_ref[f], vr_ref[f], preferred_element_type=jnp.float32)
        e_ref[f] = ef
        dsl = ef[:, f * _K:(f + 1) * _K]
        dterm = jnp.sum(dsl * dsl, axis=1)
        diag = dterm if diag is None else diag + dterm

    s = None
    for f in range(_F):
        ef = e_ref[f]
        tf = e_ref[:, :, f * _K:(f + 1) * _K]
        tf = jnp.swapaxes(tf, 0, 1).reshape(ef.shape[0], _GK)
        term = jnp.sum(ef * tf, axis=1)
        s = term if s is None else s + term

    o_ref[...] = (lin + 0.5 * (s - diag))[:, None]


def kernel(inputs, w, v):
    b = inputs.shape[0]
    bt = 256
    grid = b // bt

    xr = inputs.reshape(b, _F, _D)
    xp = jnp.pad(xr, ((0, 0), (0, 0), (0, _DP - _D))).transpose(1, 0, 2)
    vr = v.reshape(_F, _F, _D, _K).transpose(1, 2, 0, 3).reshape(_F, _D, _GK)
    vr = jnp.pad(vr, ((0, 0), (0, _DP - _D), (0, 0)))
    wp = jnp.pad(w.reshape(_F, _D), ((0, 0), (0, _DP - _D))).reshape(_F, 1, _DP)

    out = pl.pallas_call(
        _ffm_block,
        grid=(grid,),
        in_specs=[
            pl.BlockSpec((_F, bt, _DP), lambda i: (0, i, 0)),
            pl.BlockSpec((_F, _DP, _GK), lambda i: (0, 0, 0)),
            pl.BlockSpec((_F, 1, _DP), lambda i: (0, 0, 0)),
        ],
        out_specs=pl.BlockSpec((bt, 1), lambda i: (i, 0)),
        out_shape=jax.ShapeDtypeStruct((b, 1), jnp.float32),
        scratch_shapes=[pltpu.VMEM((_F, bt, _GK), jnp.float32)],
    )(xp, vr, wp)
    return out
